# Initial kernel scaffold; baseline (speedup 1.0000x reference)
import jax
import jax.numpy as jnp
from jax.experimental import pallas as pl


def kernel(adgroup_id, cate_id, brand, user, timestamp, is_click, q_proba, T_adgroup, T_cate, T_brand, Wa1, ba1, Wa2, ba2, Wa3, ba3, Wu1, bu1, Wu2, bu2, Wu3, bu3):
    raise NotImplementedError("write your pallas kernel here")


# speedup vs baseline: 1.1594x; 1.1594x over previous
import functools

import jax
import jax.numpy as jnp
from jax import lax
from jax.experimental import pallas as pl
from jax.experimental.pallas import tpu as pltpu
from jax.experimental.pallas import tpu_sc as plsc

B = 4096
D = 64
_NC = 2
_NS = 16
_NW = _NC * _NS
_BPW = B // _NW

_F32 = jnp.float32


def _dot_t(a, b):
    return lax.dot_general(a, b, (((1,), (1,)), ((), ())),
                           preferred_element_type=_F32)


def _dot(a, b):
    return lax.dot_general(a, b, (((1,), (0,)), ((), ())),
                           preferred_element_type=_F32)


def _silu(x):
    return x / (1.0 + jnp.exp(-x))


def _l2norm(x):
    n = jnp.sqrt(jnp.sum(x * x, axis=-1, keepdims=True))
    return x / jnp.maximum(n, 1e-16)


def _sc_gather3(idx_a, idx_c, idx_b, T_a3, T_c3, T_b3):
    mesh = plsc.VectorSubcoreMesh(core_axis_name="c", subcore_axis_name="s")
    out_t = jax.ShapeDtypeStruct((B, 8, D), _F32)

    @functools.partial(
        pl.kernel,
        out_type=[out_t, out_t, out_t],
        mesh=mesh,
        scratch_types=[
            pltpu.VMEM((_BPW,), jnp.int32),
            pltpu.VMEM((_BPW,), jnp.int32),
            pltpu.VMEM((_BPW, 8, D), _F32),
            pltpu.SemaphoreType.DMA,
        ],
    )
    def gather_k(ia_h, ic_h, ib_h, Ta_h, Tc_h, Tb_h, oa_h, oc_h, ob_h,
                 iv_raw, iv_g, rv, sem):
        wid = lax.axis_index("s") * _NC + lax.axis_index("c")
        base = wid * _BPW
        for idx_h, T_h, o_h in ((ia_h, Ta_h, oa_h), (ic_h, Tc_h, oc_h),
                                (ib_h, Tb_h, ob_h)):
            pltpu.sync_copy(idx_h.at[pl.ds(base, _BPW)], iv_raw)
            for i in range(_BPW // 16):
                sl = pl.ds(i * 16, 16)
                iv_g[sl] = lax.shift_right_logical(iv_raw[sl], 3)
            pltpu.async_copy(T_h.at[iv_g], rv, sem).wait()
            pltpu.sync_copy(rv, o_h.at[pl.ds(base, _BPW)])

    return gather_k(idx_a, idx_c, idx_b, T_a3, T_c3, T_b3)


_R_MLP = 512


def _sel_row(grp, ridx):
    s = lax.broadcasted_iota(jnp.int32, (_R_MLP, 8, 1), 1)
    r = (ridx & 7).reshape(_R_MLP, 1, 1)
    return jnp.sum(jnp.where(s == r, grp, 0.0), axis=1)


def _ad_mlp_body(ga, gc, gb, ra, rc, rb, w1a, w1c, w1b, b1, w2, b2, w3, b3,
                 out):
    ea = _sel_row(ga[...], ra[...])
    ec = _sel_row(gc[...], rc[...])
    eb = _sel_row(gb[...], rb[...])
    h = _dot_t(ea, w1a[...]) + _dot_t(ec, w1c[...]) \
        + _dot_t(eb, w1b[...]) + b1[...]
    h = _silu(h)
    h = _silu(_dot_t(h, w2[...]) + b2[...])
    h = _dot_t(h, w3[...]) + b3[...]
    out[...] = _l2norm(_l2norm(h))


def _ad_mlp(ga, gc, gb, ia, ic, ib, Wa1, ba1, Wa2, ba2, Wa3, ba3):
    full = lambda shape: pl.BlockSpec(shape, lambda i: (0,) * len(shape))
    row3 = pl.BlockSpec((_R_MLP, 8, D), lambda i: (i, 0, 0))
    col = pl.BlockSpec((_R_MLP, 1), lambda i: (i, 0))
    return pl.pallas_call(
        _ad_mlp_body,
        grid=(B // _R_MLP,),
        in_specs=[
            row3, row3, row3, col, col, col,
            full((256, D)), full((256, D)), full((256, D)), full((1, 256)),
            full((128, 256)), full((1, 128)),
            full((64, 128)), full((1, 64)),
        ],
        out_specs=pl.BlockSpec((_R_MLP, D), lambda i: (i, 0)),
        out_shape=jax.ShapeDtypeStruct((B, D), _F32),
    )(ga, gc, gb, ia.reshape(B, 1), ic.reshape(B, 1), ib.reshape(B, 1),
      Wa1[:, 0:D], Wa1[:, D:2 * D], Wa1[:, 2 * D:3 * D],
      ba1.reshape(1, -1), Wa2, ba2.reshape(1, -1), Wa3, ba3.reshape(1, -1))


_R_LOSS = 256
_NB_LOSS = B // _R_LOSS


def _loss_body(ad_emb, u_row, ts_row, clk_row, ids_row, qp_row,
               u_col, ts_col, clk_col, ids_col,
               w1, b1, w2, b2, w3, b3, out, acc):
    i = pl.program_id(0)

    @pl.when(i == 0)
    def _init():
        acc[0] = 0.0
        acc[1] = 0.0

    ad = ad_emb[...]
    clicked = clk_row[...] == 1
    same_user = u_col[...] == u_row[...]
    causal = ts_col[...] > ts_row[...]
    maskf = (clicked & same_user & causal).astype(_F32)
    msum = jnp.sum(maskf, axis=1, keepdims=True)
    hist = _dot(maskf, ad) / (msum + 1e-16)
    hist_n2 = jnp.sum(hist * hist, axis=1, keepdims=True)
    g = _silu(_dot_t(hist, w1[...]) + b1[...])
    g = _silu(_dot_t(g, w2[...]) + b2[...])
    g = _dot_t(g, w3[...]) + b3[...]
    x = _l2norm(_l2norm(g))
    user_emb = jnp.where(hist_n2 == 0.0, 0.0, x)
    u_n2 = jnp.sum(user_emb * user_emb, axis=1, keepdims=True)
    validf = ((clk_col[...] == 1) & (u_n2 != 0.0)).astype(_F32)
    logits = _dot_t(user_emb, ad) - jnp.log(qp_row[...])
    cols = lax.broadcasted_iota(jnp.int32, (_R_LOSS, B), 1)
    rows = lax.broadcasted_iota(jnp.int32, (_R_LOSS, B), 0) + i * _R_LOSS
    eye = cols == rows
    acc_hits = (ids_col[...] == ids_row[...]) & jnp.logical_not(eye)
    logits = jnp.where(acc_hits, -1e9, logits)
    m = jnp.max(logits, axis=1, keepdims=True)
    lse = m + jnp.log(jnp.sum(jnp.exp(logits - m), axis=1, keepdims=True))
    diag = jnp.sum(jnp.where(eye, logits, 0.0), axis=1, keepdims=True)
    pos_logp = diag - lse
    acc[0] += jnp.sum(pos_logp * validf)
    acc[1] += jnp.sum(validf)

    @pl.when(i == _NB_LOSS - 1)
    def _fin():
        out[...] = jnp.broadcast_to(-acc[0] / (acc[1] + 1e-16), (1, 1))


def _loss(ad_emb, user, timestamp, is_click, ad_ids, q_proba,
          Wu1, bu1, Wu2, bu2, Wu3, bu3):
    full = lambda shape: pl.BlockSpec(shape, lambda i: (0, 0))
    col = pl.BlockSpec((_R_LOSS, 1), lambda i: (i, 0))
    u_row = user.reshape(1, B)
    ts_row = timestamp.reshape(1, B)
    clk_row = is_click.reshape(1, B)
    ids_row = ad_ids.reshape(1, B)
    out = pl.pallas_call(
        _loss_body,
        grid=(_NB_LOSS,),
        in_specs=[
            full((B, D)),
            full((1, B)), full((1, B)), full((1, B)), full((1, B)),
            full((1, B)),
            col, col, col, col,
            full((256, D)), full((1, 256)),
            full((128, 256)), full((1, 128)),
            full((64, 128)), full((1, 64)),
        ],
        out_specs=full((1, 1)),
        out_shape=jax.ShapeDtypeStruct((1, 1), _F32),
        scratch_shapes=[pltpu.SMEM((2,), _F32)],
    )(ad_emb, u_row, ts_row, clk_row, ids_row, q_proba.reshape(1, B),
      u_row.reshape(B, 1), ts_row.reshape(B, 1), clk_row.reshape(B, 1),
      ids_row.reshape(B, 1),
      Wu1, bu1.reshape(1, -1), Wu2, bu2.reshape(1, -1),
      Wu3, bu3.reshape(1, -1))
    return out[0, 0]


def kernel(adgroup_id, cate_id, brand, user, timestamp, is_click, q_proba,
           T_adgroup, T_cate, T_brand, Wa1, ba1, Wa2, ba2, Wa3, ba3,
           Wu1, bu1, Wu2, bu2, Wu3, bu3):
    ia = adgroup_id.reshape(B).astype(jnp.int32)
    ic = cate_id.reshape(B).astype(jnp.int32)
    ib = brand.reshape(B).astype(jnp.int32)
    ga = jnp.take(T_adgroup.reshape(-1, 8, D), ia >> 3, axis=0)
    gc = jnp.take(T_cate.reshape(-1, 8, D), ic >> 3, axis=0)
    gb = jnp.take(T_brand.reshape(-1, 8, D), ib >> 3, axis=0)
    ad_emb = _ad_mlp(ga, gc, gb, ia, ic, ib, Wa1, ba1, Wa2, ba2, Wa3, ba3)
    return _loss(ad_emb, user.reshape(B), timestamp, is_click.astype(jnp.int32),
                 ia, q_proba, Wu1, bu1, Wu2, bu2, Wu3, bu3)
